# baseline (speedup 1.0000x reference)
import numpy as np

import jax
import jax.numpy as jnp
from jax.experimental import pallas as pl
from jax.experimental.pallas import tpu as pltpu

_K = 3
_Q = 4
_QP = 64


def _shift_mats(w):
    s = np.zeros((_K, w, w), np.float32)
    b = np.arange(w)
    for j in range(_K):
        s[j, (b + j - 1) % w, b] = 1.0
    return s


def _col_pool_select_q(c1, W):
    wq = W // _Q
    m = np.zeros((_Q * c1 * wq, c1 * (W // 2)), np.float32)
    for q in range(_Q):
        for co in range(c1):
            for wt in range(0, wq, 2):
                m[q * c1 * wq + co * wq + wt,
                  co * (W // 2) + (q * wq + wt) // 2] = 1.0
    return m


def _col_pool_select(c, w):
    m = np.zeros((c * w, c * (w // 2)), np.float32)
    q = np.arange(w // 2)
    for co in range(c):
        m[co * w + 2 * q, co * (w // 2) + q] = 1.0
    return m


def _fwd_kernel(Nb, H, W, cin, c1, c2, ncls,
                x_ref, a1_ref, a2_ref, s1_ref, s2_ref, wfc_ref,
                b1_ref, b2_ref, bfc_ref, out_ref):
    f32 = jnp.float32
    bf16 = jnp.bfloat16
    h2, w2 = H // 2, W // 2
    h3, w3 = h2 // 2, w2 // 2
    wq = W // _Q
    nc2 = c2 * w2

    xb = x_ref[...].reshape(H * Nb, _Q * _QP)
    z1 = jnp.zeros((Nb, _Q * _QP), bf16)
    xf = jnp.concatenate([z1, xb, z1], axis=0)
    yqs = []
    for q in range(_Q):
        xq = xf[:, q * _QP:(q + 1) * _QP]
        xc = jnp.concatenate(
            [xq[d * Nb:(H + d) * Nb] for d in range(_K)], axis=1)
        yqs.append(jnp.dot(xc, a1_ref[q], preferred_element_type=f32))
    y = (jnp.concatenate(yqs, axis=1) + b1_ref[...]).astype(bf16)

    y3 = y.reshape(h2, 2, Nb, _Q * c1 * wq)
    yr = jnp.maximum(y3[:, 0], y3[:, 1])
    yc = jnp.maximum(yr, pltpu.roll(yr, _Q * c1 * wq - 1, axis=2))
    p1 = jnp.maximum(
        jnp.dot(yc.reshape(h2 * Nb, _Q * c1 * wq), s1_ref[...],
                preferred_element_type=f32), 0.0).astype(bf16)

    z2 = jnp.zeros((Nb, c1 * w2), bf16)
    pf = jnp.concatenate([z2, p1, z2], axis=0)
    pc = jnp.concatenate(
        [pf[d * Nb:(h2 + d) * Nb] for d in range(_K)], axis=1)
    y2 = (jnp.dot(pc, a2_ref[...], preferred_element_type=f32)
          + b2_ref[...]).astype(bf16)

    y23 = y2.reshape(h3, 2, Nb, nc2)
    y2r = jnp.maximum(y23[:, 0], y23[:, 1])
    y2c = jnp.maximum(y2r, pltpu.roll(y2r, nc2 - 1, axis=2))
    p2 = jnp.maximum(
        jnp.dot(y2c.reshape(h3 * Nb, nc2), s2_ref[...],
                preferred_element_type=f32), 0.0).astype(bf16)

    p23 = p2.reshape(h3, Nb, c2 * w3)
    acc = jnp.dot(p23[0], wfc_ref[0], preferred_element_type=f32)
    for r in range(1, h3):
        acc = acc + jnp.dot(p23[r], wfc_ref[r], preferred_element_type=f32)
    out_ref[...] = acc + bfc_ref[...]


def kernel(x, W1, b1, W2, b2, Wfc2, bfc2):
    import functools
    f32 = jnp.float32
    bf16 = jnp.bfloat16
    B, cin, H, W = x.shape
    c1 = W1.shape[0]
    c2 = W2.shape[0]
    ncls = Wfc2.shape[0]
    h2, w2 = H // 2, W // 2
    h3, w3 = h2 // 2, w2 // 2
    wq = W // _Q

    Nb = 64
    while B % Nb:
        Nb //= 2
    G = B // Nb

    xh = x.astype(bf16)
    wins = []
    for q in range(_Q):
        lo = (q * wq - 1) % W
        hi = lo + wq + 2
        if hi <= W:
            wins.append(xh[..., lo:hi])
        else:
            wins.append(jnp.concatenate(
                [xh[..., lo:], xh[..., :hi - W]], axis=-1))
    xw = jnp.stack(wins, axis=3)
    xw = xw.transpose(2, 0, 3, 1, 4).reshape(H, B, _Q, cin * (wq + 2))
    xw = jnp.pad(xw, ((0, 0), (0, 0), (0, 0), (0, _QP - cin * (wq + 2))))
    xt = xw.reshape(H, B, _Q * _QP)

    T = np.zeros((_K, wq + 2, wq), np.float32)
    for dj in range(_K):
        T[dj, np.arange(wq) + dj, np.arange(wq)] = 1.0
    A1q = jnp.einsum('ocdj,jkt->dckot', W1.astype(f32), jnp.asarray(T))
    A1q = jnp.pad(A1q.reshape(_K, cin * (wq + 2), c1 * wq),
                  ((0, 0), (0, _QP - cin * (wq + 2)), (0, 0)))
    A1q = A1q.reshape(1, _K * _QP, c1 * wq).astype(bf16)
    A1 = jnp.broadcast_to(A1q, (_Q, _K * _QP, c1 * wq))

    sm2 = _shift_mats(w2)
    A2 = jnp.einsum('ocdj,jab->dcaob', W2.astype(f32),
                    jnp.asarray(sm2)).reshape(_K * c1 * w2, c2 * w2).astype(bf16)

    S1 = jnp.asarray(_col_pool_select_q(c1, W), bf16)
    S2 = jnp.asarray(_col_pool_select(c2, w2), bf16)
    Wfc3 = (Wfc2.astype(f32).reshape(ncls, c2, h3, w3)
            .transpose(2, 1, 3, 0).reshape(h3, c2 * w3, ncls).astype(bf16))
    b1r = jnp.tile(jnp.repeat(b1.astype(f32), wq), _Q).reshape(1, _Q * c1 * wq)
    b2r = jnp.repeat(b2.astype(f32), w2).reshape(1, c2 * w2)
    bfc = bfc2.astype(f32).reshape(1, ncls)

    kfn = functools.partial(_fwd_kernel, Nb, H, W, cin, c1, c2, ncls)

    def const_spec(a):
        nd = a.ndim
        return pl.BlockSpec(a.shape, lambda g, _n=nd: (0,) * _n)

    out = pl.pallas_call(
        kfn,
        out_shape=jax.ShapeDtypeStruct((B, ncls), f32),
        grid=(G,),
        in_specs=[
            pl.BlockSpec((H, Nb, _Q * _QP), lambda g: (0, g, 0)),
            const_spec(A1), const_spec(A2),
            const_spec(S1), const_spec(S2), const_spec(Wfc3),
            const_spec(b1r), const_spec(b2r), const_spec(bfc),
        ],
        out_specs=pl.BlockSpec((Nb, ncls), lambda g: (g, 0)),
        compiler_params=pltpu.CompilerParams(
            dimension_semantics=("parallel",)),
    )(xt, A1, A2, S1, S2, Wfc3, b1r, b2r, bfc)
    return out


# speedup vs baseline: 1.5204x; 1.5204x over previous
import numpy as np

import jax
import jax.numpy as jnp
from jax.experimental import pallas as pl
from jax.experimental.pallas import tpu as pltpu

_K = 3
_Q = 4
_QP = 64


def _shift_mats(w):
    s = np.zeros((_K, w, w), np.float32)
    b = np.arange(w)
    for j in range(_K):
        s[j, (b + j - 1) % w, b] = 1.0
    return s


def _col_pool_select_q(c1, W):
    wq = W // _Q
    m = np.zeros((_Q * c1 * wq, c1 * (W // 2)), np.float32)
    for q in range(_Q):
        for co in range(c1):
            for wt in range(0, wq, 2):
                m[q * c1 * wq + co * wq + wt,
                  co * (W // 2) + (q * wq + wt) // 2] = 1.0
    return m


def _col_pool_select(c, w):
    m = np.zeros((c * w, c * (w // 2)), np.float32)
    q = np.arange(w // 2)
    for co in range(c):
        m[co * w + 2 * q, co * (w // 2) + q] = 1.0
    return m


def _fwd_kernel(Nb, H, W, cin, c1, c2, ncls,
                x_ref, p_ref, a1_ref, a2_ref, s1_ref, s2_ref, wfc_ref,
                b1_ref, b2_ref, bfc_ref, out_ref):
    f32 = jnp.float32
    bf16 = jnp.bfloat16
    h2, w2 = H // 2, W // 2
    h3, w3 = h2 // 2, w2 // 2
    wq = W // _Q
    nc2 = c2 * w2

    xb = x_ref[...].reshape(H * Nb, cin * W)
    xw = jnp.dot(xb, p_ref[...], preferred_element_type=f32).astype(bf16)

    z1 = jnp.zeros((Nb, _Q * _QP), bf16)
    xf = jnp.concatenate([z1, xw, z1], axis=0)
    yqs = []
    for q in range(_Q):
        xq = xf[:, q * _QP:(q + 1) * _QP]
        xc = jnp.concatenate(
            [xq[d * Nb:(H + d) * Nb] for d in range(_K)], axis=1)
        yqs.append(jnp.dot(xc, a1_ref[q], preferred_element_type=f32))
    y = (jnp.concatenate(yqs, axis=1) + b1_ref[...]).astype(bf16)

    y3 = y.reshape(h2, 2, Nb, _Q * c1 * wq)
    yr = jnp.maximum(y3[:, 0], y3[:, 1])
    yc = jnp.maximum(yr, pltpu.roll(yr, _Q * c1 * wq - 1, axis=2))
    p1 = jnp.maximum(
        jnp.dot(yc.reshape(h2 * Nb, _Q * c1 * wq), s1_ref[...],
                preferred_element_type=f32), 0.0).astype(bf16)

    z2 = jnp.zeros((Nb, c1 * w2), bf16)
    pf = jnp.concatenate([z2, p1, z2], axis=0)
    pc = jnp.concatenate(
        [pf[d * Nb:(h2 + d) * Nb] for d in range(_K)], axis=1)
    y2 = (jnp.dot(pc, a2_ref[...], preferred_element_type=f32)
          + b2_ref[...]).astype(bf16)

    y23 = y2.reshape(h3, 2, Nb, nc2)
    y2r = jnp.maximum(y23[:, 0], y23[:, 1])
    y2c = jnp.maximum(y2r, pltpu.roll(y2r, nc2 - 1, axis=2))
    p2 = jnp.maximum(
        jnp.dot(y2c.reshape(h3 * Nb, nc2), s2_ref[...],
                preferred_element_type=f32), 0.0).astype(bf16)

    p23 = p2.reshape(h3, Nb, c2 * w3)
    acc = jnp.dot(p23[0], wfc_ref[0], preferred_element_type=f32)
    for r in range(1, h3):
        acc = acc + jnp.dot(p23[r], wfc_ref[r], preferred_element_type=f32)
    out_ref[...] = acc + bfc_ref[...]


def kernel(x, W1, b1, W2, b2, Wfc2, bfc2):
    import functools
    f32 = jnp.float32
    bf16 = jnp.bfloat16
    B, cin, H, W = x.shape
    c1 = W1.shape[0]
    c2 = W2.shape[0]
    ncls = Wfc2.shape[0]
    h2, w2 = H // 2, W // 2
    h3, w3 = h2 // 2, w2 // 2
    wq = W // _Q

    Nb = 64
    while B % Nb:
        Nb //= 2
    G = B // Nb

    xt = x.astype(bf16).transpose(2, 0, 1, 3).reshape(H, B, cin * W)

    Pm = np.zeros((cin * W, _Q * _QP), np.float32)
    for q in range(_Q):
        for ci in range(cin):
            for k in range(wq + 2):
                Pm[ci * W + (q * wq + k - 1) % W,
                   q * _QP + ci * (wq + 2) + k] = 1.0
    P = jnp.asarray(Pm, bf16)

    T = np.zeros((_K, wq + 2, wq), np.float32)
    for dj in range(_K):
        T[dj, np.arange(wq) + dj, np.arange(wq)] = 1.0
    A1q = jnp.einsum('ocdj,jkt->dckot', W1.astype(f32), jnp.asarray(T))
    A1q = jnp.pad(A1q.reshape(_K, cin * (wq + 2), c1 * wq),
                  ((0, 0), (0, _QP - cin * (wq + 2)), (0, 0)))
    A1q = A1q.reshape(1, _K * _QP, c1 * wq).astype(bf16)
    A1 = jnp.broadcast_to(A1q, (_Q, _K * _QP, c1 * wq))

    sm2 = _shift_mats(w2)
    A2 = jnp.einsum('ocdj,jab->dcaob', W2.astype(f32),
                    jnp.asarray(sm2)).reshape(_K * c1 * w2, c2 * w2).astype(bf16)

    S1 = jnp.asarray(_col_pool_select_q(c1, W), bf16)
    S2 = jnp.asarray(_col_pool_select(c2, w2), bf16)
    Wfc3 = (Wfc2.astype(f32).reshape(ncls, c2, h3, w3)
            .transpose(2, 1, 3, 0).reshape(h3, c2 * w3, ncls).astype(bf16))
    b1r = jnp.tile(jnp.repeat(b1.astype(f32), wq), _Q).reshape(1, _Q * c1 * wq)
    b2r = jnp.repeat(b2.astype(f32), w2).reshape(1, c2 * w2)
    bfc = bfc2.astype(f32).reshape(1, ncls)

    kfn = functools.partial(_fwd_kernel, Nb, H, W, cin, c1, c2, ncls)

    def const_spec(a):
        nd = a.ndim
        return pl.BlockSpec(a.shape, lambda g, _n=nd: (0,) * _n)

    out = pl.pallas_call(
        kfn,
        out_shape=jax.ShapeDtypeStruct((B, ncls), f32),
        grid=(G,),
        in_specs=[
            pl.BlockSpec((H, Nb, cin * W), lambda g: (0, g, 0)),
            const_spec(P), const_spec(A1), const_spec(A2),
            const_spec(S1), const_spec(S2), const_spec(Wfc3),
            const_spec(b1r), const_spec(b2r), const_spec(bfc),
        ],
        out_specs=pl.BlockSpec((Nb, ncls), lambda g: (g, 0)),
        compiler_params=pltpu.CompilerParams(
            dimension_semantics=("parallel",)),
    )(xt, P, A1, A2, S1, S2, Wfc3, b1r, b2r, bfc)
    return out
